# baseline (speedup 1.0000x reference)
import functools

import jax
import jax.numpy as jnp
from jax import lax
from jax.experimental import pallas as pl
from jax.experimental.pallas import tpu as pltpu
from jax.experimental.pallas import tpu_sc as plsc


def _make_sc_kernel(B, T, U, I):
    info = plsc.get_sparse_core_info()
    NC, NS, L = info.num_cores, info.num_subcores, info.num_lanes
    NW = NC * NS
    assert B % (8 * NW) == 0
    BPW = B // NW
    UI = U * I
    G = BPW // L

    mesh = plsc.VectorSubcoreMesh(core_axis_name="c", subcore_axis_name="s")

    @functools.partial(
        pl.kernel,
        mesh=mesh,
        out_type=jax.ShapeDtypeStruct((B,), jnp.float32),
        scratch_types=[
            pltpu.VMEM((BPW,), jnp.int32),
            pltpu.VMEM((BPW,), jnp.int32),
            pltpu.VMEM((BPW,), jnp.int32),
            pltpu.VMEM((BPW,), jnp.int32),
            pltpu.VMEM((T * BPW,), jnp.int32),
            pltpu.VMEM((T * BPW,), jnp.float32),
            pltpu.VMEM((BPW,), jnp.float32),
            pltpu.SemaphoreType.DMA,
        ],
    )
    def sc_kernel(uid_h, iid_h, tid_h, qos_h, out_h,
                  uid_v, iid_v, tid_v, base_v, gidx_v, vals_v, out_v, sem):
        wid = lax.axis_index("s") * NC + lax.axis_index("c")
        q0 = wid * BPW
        pltpu.sync_copy(uid_h.at[pl.ds(q0, BPW)], uid_v)
        pltpu.sync_copy(iid_h.at[pl.ds(q0, BPW)], iid_v)
        pltpu.sync_copy(tid_h.at[pl.ds(q0, BPW)], tid_v)

        for g in range(G):
            sl = pl.ds(g * L, L)
            base_v[sl] = uid_v[sl] * I + iid_v[sl]

        def fill_t(t, _):
            off = t * UI
            for g in range(G):
                gidx_v[pl.ds(t * BPW + g * L, L)] = base_v[pl.ds(g * L, L)] + off
            return 0

        lax.fori_loop(0, T, fill_t, 0)

        pltpu.async_copy(qos_h.at[gidx_v], vals_v, sem).wait()

        def per_group(g, _):
            jl0 = g * L
            tv = tid_v[pl.ds(jl0, L)]

            def over_t(t, carry):
                s, c, cur = carry
                v = vals_v[pl.ds(t * BPW + jl0, L)]
                return (s + v, c + jnp.where(v > 0, 1.0, 0.0),
                        jnp.where(t == tv, v, cur))

            zero = jnp.zeros((L,), jnp.float32)
            s, c, curr = lax.fori_loop(0, T, over_t, (zero, zero, zero))
            s_o = s - curr
            c_o = c - jnp.where(curr > 0, 1.0, 0.0)
            out_v[pl.ds(jl0, L)] = jnp.where(c_o > 0, s_o / c_o, 0.0)
            return 0

        lax.fori_loop(0, G, per_group, 0)
        pltpu.sync_copy(out_v, out_h.at[pl.ds(q0, BPW)])

    return sc_kernel


def kernel(user_id, item_id, time_id, qos_tensor):
    T, U, I = qos_tensor.shape
    B = user_id.shape[0]
    uid = user_id.astype(jnp.int32)
    iid = item_id.astype(jnp.int32)
    tid = time_id.astype(jnp.int32)
    qflat = qos_tensor.reshape(T * U * I)
    return _make_sc_kernel(B, T, U, I)(uid, iid, tid, qflat)


# speedup vs baseline: 1.0007x; 1.0007x over previous
import functools

import jax
import jax.numpy as jnp
from jax import lax
from jax.experimental import pallas as pl
from jax.experimental.pallas import tpu as pltpu
from jax.experimental.pallas import tpu_sc as plsc


def _make_sc_kernel(B, T, U, I):
    info = plsc.get_sparse_core_info()
    NC, NS, L = info.num_cores, info.num_subcores, info.num_lanes
    NW = NC * NS
    assert B % (8 * NW) == 0
    BPW = B // NW
    UI = U * I
    G = BPW // L

    mesh = plsc.VectorSubcoreMesh(core_axis_name="c", subcore_axis_name="s")

    @functools.partial(
        pl.kernel,
        mesh=mesh,
        out_type=jax.ShapeDtypeStruct((B,), jnp.float32),
        scratch_types=[
            pltpu.VMEM((BPW,), jnp.int32),
            pltpu.VMEM((BPW,), jnp.int32),
            pltpu.VMEM((BPW,), jnp.int32),
            pltpu.VMEM((BPW,), jnp.int32),
            pltpu.VMEM((T * BPW,), jnp.int32),
            pltpu.VMEM((T * BPW,), jnp.float32),
            pltpu.VMEM((BPW,), jnp.float32),
        ] + [pltpu.SemaphoreType.DMA] * 8,
    )
    def sc_kernel(uid_h, iid_h, tid_h, qos_h, out_h,
                  uid_v, iid_v, tid_v, base_v, gidx_v, vals_v, out_v, *sems):
        wid = lax.axis_index("s") * NC + lax.axis_index("c")
        q0 = wid * BPW
        pltpu.sync_copy(uid_h.at[pl.ds(q0, BPW)], uid_v)
        pltpu.sync_copy(iid_h.at[pl.ds(q0, BPW)], iid_v)
        pltpu.sync_copy(tid_h.at[pl.ds(q0, BPW)], tid_v)

        for g in range(G):
            sl = pl.ds(g * L, L)
            base_v[sl] = uid_v[sl] * I + iid_v[sl]

        def fill_t(t, _):
            off = t * UI
            for g in range(G):
                gidx_v[pl.ds(t * BPW + g * L, L)] = base_v[pl.ds(g * L, L)] + off
            return 0

        lax.fori_loop(0, T, fill_t, 0)

        NCH = 8
        CH = (T * BPW) // NCH
        copies = [
            pltpu.async_copy(
                qos_h.at[gidx_v.at[pl.ds(k * CH, CH)]],
                vals_v.at[pl.ds(k * CH, CH)],
                sems[k],
            )
            for k in range(NCH)
        ]
        for c in copies:
            c.wait()

        def per_group(g, _):
            jl0 = g * L
            tv = tid_v[pl.ds(jl0, L)]

            def over_t(t, carry):
                s, c, cur = carry
                v = vals_v[pl.ds(t * BPW + jl0, L)]
                return (s + v, c + jnp.where(v > 0, 1.0, 0.0),
                        jnp.where(t == tv, v, cur))

            zero = jnp.zeros((L,), jnp.float32)
            s, c, curr = lax.fori_loop(0, T, over_t, (zero, zero, zero))
            s_o = s - curr
            c_o = c - jnp.where(curr > 0, 1.0, 0.0)
            out_v[pl.ds(jl0, L)] = jnp.where(c_o > 0, s_o / c_o, 0.0)
            return 0

        lax.fori_loop(0, G, per_group, 0)
        pltpu.sync_copy(out_v, out_h.at[pl.ds(q0, BPW)])

    return sc_kernel


def kernel(user_id, item_id, time_id, qos_tensor):
    T, U, I = qos_tensor.shape
    B = user_id.shape[0]
    uid = user_id.astype(jnp.int32)
    iid = item_id.astype(jnp.int32)
    tid = time_id.astype(jnp.int32)
    qflat = qos_tensor.reshape(T * U * I)
    return _make_sc_kernel(B, T, U, I)(uid, iid, tid, qflat)


# speedup vs baseline: 8.7462x; 8.7404x over previous
import functools

import jax
import jax.numpy as jnp
from jax import lax
from jax.experimental import pallas as pl
from jax.experimental.pallas import tpu as pltpu
from jax.experimental.pallas import tpu_sc as plsc

_IP = 5888


def _tc_tables(qos):
    T, U, I = qos.shape
    UB = 8
    NU = (U + UB - 1) // UB

    def body(q_ref, s_ref, c_ref):
        x = q_ref[...]
        s = jnp.sum(x, axis=0)
        c = jnp.sum(jnp.where(x > 0, 1.0, 0.0), axis=0)
        for r in range(UB):
            s_ref[pl.ds(r * _IP, I)] = s[r]
            c_ref[pl.ds(r * _IP, I)] = c[r]

    out_len = NU * UB * _IP
    out_sd = jax.ShapeDtypeStruct((out_len,), jnp.float32)
    return pl.pallas_call(
        body,
        grid=(NU,),
        in_specs=[pl.BlockSpec((T, UB, I), lambda u: (0, u, 0))],
        out_specs=[pl.BlockSpec((UB * _IP,), lambda u: (u,)),
                   pl.BlockSpec((UB * _IP,), lambda u: (u,))],
        out_shape=[out_sd, out_sd],
    )(qos)


def _make_sc_kernel(B, T, U, I):
    info = plsc.get_sparse_core_info()
    NC, NS, L = info.num_cores, info.num_subcores, info.num_lanes
    NW = NC * NS
    assert B % (8 * NW) == 0
    BPW = B // NW
    NCHK = 2
    QC = BPW // NCHK
    NF = 16

    mesh = plsc.VectorSubcoreMesh(core_axis_name="c", subcore_axis_name="s")

    @functools.partial(
        pl.kernel,
        mesh=mesh,
        out_type=jax.ShapeDtypeStruct((B,), jnp.float32),
        compiler_params=pltpu.CompilerParams(needs_layout_passes=False),
        scratch_types=[
            pltpu.VMEM((QC,), jnp.int32),
            pltpu.VMEM((QC,), jnp.int32),
            pltpu.VMEM((QC,), jnp.int32),
            pltpu.VMEM((QC,), jnp.int32),
            pltpu.VMEM((QC,), jnp.float32),
            pltpu.VMEM((QC,), jnp.float32),
            pltpu.VMEM((NF, 8, 128), jnp.float32),
            pltpu.VMEM((QC,), jnp.float32),
            pltpu.VMEM((QC,), jnp.float32),
            pltpu.SemaphoreType.DMA,
            pltpu.SemaphoreType.DMA,
        ] + [pltpu.SemaphoreType.DMA] * NF,
    )
    def sc_kernel(uid_h, iid_h, tid_h, sum_h, cnt_h, qos_h, out_h,
                  uid_v, iid_v, tid_v, pix_v,
                  s_v, c_v, tiles_v, cur_v, out_v, sem_s, sem_c, *sems):
        wid = lax.axis_index("s") * NC + lax.axis_index("c")
        q0 = wid * BPW

        for ch in range(NCHK):
            c0 = q0 + ch * QC
            pltpu.sync_copy(uid_h.at[pl.ds(c0, QC)], uid_v)
            pltpu.sync_copy(iid_h.at[pl.ds(c0, QC)], iid_v)
            pltpu.sync_copy(tid_h.at[pl.ds(c0, QC)], tid_v)

            for g in range(QC // L):
                sl = pl.ds(g * L, L)
                pix_v[sl] = uid_v[sl] * _IP + iid_v[sl]

            cp_s = pltpu.async_copy(sum_h.at[pix_v], s_v, sem_s)
            cp_c = pltpu.async_copy(cnt_h.at[pix_v], c_v, sem_c)

            def wave(w, _):
                w0 = w * NF
                wsl = pl.ds(w0, L)
                uvec = uid_v[wsl]
                ivec = iid_v[wsl]
                tvec = tid_v[wsl]
                cps = []
                for c in range(NF):
                    u0 = pl.multiple_of(uvec[c] & -8, 8)
                    i0 = pl.multiple_of(ivec[c] & -128, 128)
                    cps.append(pltpu.async_copy(
                        qos_h.at[tvec[c], pl.ds(u0, 8), pl.ds(i0, 128)],
                        tiles_v.at[c], sems[c]))
                for cp in cps:
                    cp.wait()
                sl = pl.ds(w0, L)
                cur_v[sl] = plsc.load_gather(
                    tiles_v,
                    [lax.iota(jnp.int32, L), uid_v[sl] & 7, iid_v[sl] & 127])
                return 0

            lax.fori_loop(0, QC // NF, wave, 0)
            cp_s.wait()
            cp_c.wait()

            for g in range(QC // L):
                sl = pl.ds(g * L, L)
                s = s_v[sl]
                c = c_v[sl]
                cur = cur_v[sl]
                s_o = s - cur
                c_o = c - jnp.where(cur > 0, 1.0, 0.0)
                out_v[sl] = jnp.where(c_o > 0, s_o / c_o, 0.0)

            pltpu.sync_copy(out_v, out_h.at[pl.ds(c0, QC)])

    return sc_kernel


def kernel(user_id, item_id, time_id, qos_tensor):
    T, U, I = qos_tensor.shape
    B = user_id.shape[0]
    uid = user_id.astype(jnp.int32)
    iid = item_id.astype(jnp.int32)
    tid = time_id.astype(jnp.int32)
    sum_tab, cnt_tab = _tc_tables(qos_tensor)
    return _make_sc_kernel(B, T, U, I)(uid, iid, tid, sum_tab, cnt_tab,
                                       qos_tensor)


# speedup vs baseline: 8.8543x; 1.0124x over previous
import functools

import jax
import jax.numpy as jnp
from jax import lax
from jax.experimental import pallas as pl
from jax.experimental.pallas import tpu as pltpu
from jax.experimental.pallas import tpu_sc as plsc

_IP = 5888


def _tc_tables(qos):
    T, U, I = qos.shape

    def body(q_ref, s_ref, c_ref, s_scr, c_scr):
        t = pl.program_id(0)
        x = q_ref[0]
        nz = jnp.where(x > 0, 1.0, 0.0)

        @pl.when(t == 0)
        def _():
            s_scr[...] = x
            c_scr[...] = nz

        @pl.when(t != 0)
        def _():
            s_scr[...] = s_scr[...] + x
            c_scr[...] = c_scr[...] + nz

        @pl.when(t == T - 1)
        def _():
            def row(r, _):
                off = pl.multiple_of(r * _IP, 128)
                s_ref[pl.ds(off, I)] = s_scr[r]
                c_ref[pl.ds(off, I)] = c_scr[r]
                return 0

            lax.fori_loop(0, U, row, 0)

    out_sd = jax.ShapeDtypeStruct((U * _IP,), jnp.float32)
    return pl.pallas_call(
        body,
        grid=(T,),
        compiler_params=pltpu.CompilerParams(
            vmem_limit_bytes=100 * 1024 * 1024),
        in_specs=[pl.BlockSpec((1, U, I), lambda t: (t, 0, 0))],
        out_specs=[pl.BlockSpec((U * _IP,), lambda t: (0,)),
                   pl.BlockSpec((U * _IP,), lambda t: (0,))],
        out_shape=[out_sd, out_sd],
        scratch_shapes=[pltpu.VMEM((U, I), jnp.float32),
                        pltpu.VMEM((U, I), jnp.float32)],
    )(qos)


def _make_sc_curr(B, T, U, I):
    info = plsc.get_sparse_core_info()
    NC, NS, L = info.num_cores, info.num_subcores, info.num_lanes
    NW = NC * NS
    assert B % (8 * NW) == 0
    BPW = B // NW
    NF = 16

    mesh = plsc.VectorSubcoreMesh(core_axis_name="c", subcore_axis_name="s")

    @functools.partial(
        pl.kernel,
        mesh=mesh,
        out_type=jax.ShapeDtypeStruct((B,), jnp.float32),
        compiler_params=pltpu.CompilerParams(needs_layout_passes=False),
        scratch_types=[
            pltpu.VMEM((BPW,), jnp.int32),
            pltpu.VMEM((BPW,), jnp.int32),
            pltpu.VMEM((BPW,), jnp.int32),
            pltpu.VMEM((NF, 8, 128), jnp.float32),
            pltpu.VMEM((BPW,), jnp.float32),
        ] + [pltpu.SemaphoreType.DMA] * NF,
    )
    def sc_curr(uid_h, iid_h, tid_h, qos_h, cur_out_h,
                uid_v, iid_v, tid_v, tiles_v, cur_v, *sems):
        wid = lax.axis_index("s") * NC + lax.axis_index("c")
        q0 = wid * BPW
        pltpu.sync_copy(uid_h.at[pl.ds(q0, BPW)], uid_v)
        pltpu.sync_copy(iid_h.at[pl.ds(q0, BPW)], iid_v)
        pltpu.sync_copy(tid_h.at[pl.ds(q0, BPW)], tid_v)

        def wave(w, _):
            w0 = w * NF
            wsl = pl.ds(w0, L)
            uvec = uid_v[wsl]
            ivec = iid_v[wsl]
            tvec = tid_v[wsl]
            cps = []
            for c in range(NF):
                u0 = pl.multiple_of(uvec[c] & -8, 8)
                i0 = pl.multiple_of(ivec[c] & -128, 128)
                cps.append(pltpu.async_copy(
                    qos_h.at[tvec[c], pl.ds(u0, 8), pl.ds(i0, 128)],
                    tiles_v.at[c], sems[c]))
            for cp in cps:
                cp.wait()
            cur_v[wsl] = plsc.load_gather(
                tiles_v, [lax.iota(jnp.int32, L), uvec & 7, ivec & 127])
            return 0

        lax.fori_loop(0, BPW // NF, wave, 0)
        pltpu.sync_copy(cur_v, cur_out_h.at[pl.ds(q0, BPW)])

    return sc_curr


def _make_sc_combine(B):
    info = plsc.get_sparse_core_info()
    NC, NS, L = info.num_cores, info.num_subcores, info.num_lanes
    NW = NC * NS
    BPW = B // NW

    mesh = plsc.VectorSubcoreMesh(core_axis_name="c", subcore_axis_name="s")

    @functools.partial(
        pl.kernel,
        mesh=mesh,
        out_type=jax.ShapeDtypeStruct((B,), jnp.float32),
        compiler_params=pltpu.CompilerParams(needs_layout_passes=False),
        scratch_types=[
            pltpu.VMEM((BPW,), jnp.int32),
            pltpu.VMEM((BPW,), jnp.int32),
            pltpu.VMEM((BPW,), jnp.int32),
            pltpu.VMEM((BPW,), jnp.float32),
            pltpu.VMEM((BPW,), jnp.float32),
            pltpu.VMEM((BPW,), jnp.float32),
            pltpu.VMEM((BPW,), jnp.float32),
            pltpu.SemaphoreType.DMA,
            pltpu.SemaphoreType.DMA,
        ],
    )
    def sc_combine(uid_h, iid_h, cur_h, sum_h, cnt_h, out_h,
                   uid_v, iid_v, pix_v, s_v, c_v, cur_v, out_v,
                   sem_s, sem_c):
        wid = lax.axis_index("s") * NC + lax.axis_index("c")
        q0 = wid * BPW
        pltpu.sync_copy(uid_h.at[pl.ds(q0, BPW)], uid_v)
        pltpu.sync_copy(iid_h.at[pl.ds(q0, BPW)], iid_v)
        pltpu.sync_copy(cur_h.at[pl.ds(q0, BPW)], cur_v)

        for g in range(BPW // L):
            sl = pl.ds(g * L, L)
            pix_v[sl] = uid_v[sl] * _IP + iid_v[sl]

        cp_s = pltpu.async_copy(sum_h.at[pix_v], s_v, sem_s)
        cp_c = pltpu.async_copy(cnt_h.at[pix_v], c_v, sem_c)
        cp_s.wait()
        cp_c.wait()

        for g in range(BPW // L):
            sl = pl.ds(g * L, L)
            s = s_v[sl]
            c = c_v[sl]
            cur = cur_v[sl]
            s_o = s - cur
            c_o = c - jnp.where(cur > 0, 1.0, 0.0)
            out_v[sl] = jnp.where(c_o > 0, s_o / c_o, 0.0)

        pltpu.sync_copy(out_v, out_h.at[pl.ds(q0, BPW)])

    return sc_combine


def kernel(user_id, item_id, time_id, qos_tensor):
    T, U, I = qos_tensor.shape
    B = user_id.shape[0]
    uid = user_id.astype(jnp.int32)
    iid = item_id.astype(jnp.int32)
    tid = time_id.astype(jnp.int32)
    cur = _make_sc_curr(B, T, U, I)(uid, iid, tid, qos_tensor)
    sum_tab, cnt_tab = _tc_tables(qos_tensor)
    return _make_sc_combine(B)(uid, iid, cur, sum_tab, cnt_tab)
